# baseline (speedup 1.0000x reference)
import jax
import jax.numpy as jnp
from jax import lax
from jax.experimental import pallas as pl
from jax.experimental.pallas import tpu as pltpu
from jax.experimental.pallas import tpu_sc as plsc

N = 50000
E = 800000
D = 128
H = 64
NP = 50048
HALF = NP // 2
ACC_ROWS = HALF + 64
H2 = 32
CH = 128
EC = E // CH
ECP = 6400
DUMMY_DST = NP - 8

A_CPT = ECP // 16
A_CPG = 4
A_NG = A_CPT // A_CPG

D_CPT = ECP // 32
D_CPG = 8
D_NG = D_CPT // D_CPG


def _deg_body(dst_hbm, out_hbm, ones_v, zbuf, dblk, acc, ssem, isem):
    c = lax.axis_index("c")
    s = lax.axis_index("s")
    zero16 = jnp.zeros((16,), jnp.float32)
    one16 = jnp.ones((16,), jnp.float32)
    for k in range(CH // 16):
        ones_v[pl.ds(16 * k, 16)] = one16

    def zb(i, _):
        zbuf[pl.ds(i * 16, 16)] = zero16
        return 0

    lax.fori_loop(0, 196, zb, 0)
    tpr = NP // 16
    pltpu.sync_copy(zbuf.at[pl.ds(0, tpr)], acc.at[pl.ds(s * tpr, tpr)])
    plsc.subcore_barrier()

    start = c * (ECP // 2) + s * D_CPT

    def idx_src(g):
        return dst_hbm.at[pl.ds(start + D_CPG * g, D_CPG)]

    pltpu.sync_copy(idx_src(0), dblk.at[0])
    pltpu.async_copy(idx_src(1), dblk.at[1], isem)

    def body(g, _):
        r = lax.rem(g, 3)
        rp = lax.rem(g + 2, 3)
        for i in range(D_CPG):
            pltpu.async_copy(ones_v, acc.at[dblk.at[r, i]], ssem, add=True)

        @pl.when(g > 0)
        def _():
            for i in range(D_CPG):
                pltpu.make_async_copy(ones_v, acc.at[dblk.at[rp, i]], ssem).wait()

        @pl.when(g < D_NG - 1)
        def _():
            pltpu.make_async_copy(idx_src(g + 1), dblk.at[lax.rem(g + 1, 3)],
                                  isem).wait()

        @pl.when(g < D_NG - 2)
        def _():
            pltpu.async_copy(idx_src(g + 2), dblk.at[rp], isem)

        return 0

    lax.fori_loop(0, D_NG, body, 0)
    rl = lax.rem(D_NG - 1, 3)
    for i in range(D_CPG):
        pltpu.make_async_copy(ones_v, acc.at[dblk.at[rl, i]], ssem).wait()

    plsc.subcore_barrier()
    pltpu.sync_copy(acc.at[pl.ds(s * tpr, tpr)], zbuf.at[pl.ds(0, tpr)])
    pltpu.sync_copy(zbuf.at[pl.ds(0, tpr)],
                    out_hbm.at[pl.ds(c * NP + s * tpr, tpr)])


def _agg_body(g_hbm, src_hbm, dst_hbm, out_hbm, sblk, dblk, dloc, rows, acc,
              gsem, ssem, isem):
    c = lax.axis_index("c")
    s = lax.axis_index("s")
    base = c * HALF
    zero16 = jnp.zeros((16,), jnp.float32)

    def zb(i, _):
        for k in range(H2 // 16):
            rows[0, 0, i, pl.ds(16 * k, 16)] = zero16
        return 0

    lax.fori_loop(0, CH, zb, 0)
    nchunks = ACC_ROWS // 128

    def zc(k, _):
        m = s + 16 * k

        @pl.when(m < nchunks)
        def _():
            pltpu.sync_copy(rows.at[0, 0], acc.at[pl.ds(m * 128, 128)])

        return 0

    lax.fori_loop(0, (nchunks + 15) // 16, zc, 0)
    plsc.subcore_barrier()

    start = s * A_CPT

    def sidx(g):
        return src_hbm.at[pl.ds(start + A_CPG * g, A_CPG)]

    def didx(g):
        return dst_hbm.at[pl.ds(start + A_CPG * g, A_CPG)]

    pltpu.sync_copy(sidx(0), sblk.at[0])
    pltpu.sync_copy(didx(0), dblk.at[0])
    pltpu.async_copy(sidx(1), sblk.at[1], isem)
    pltpu.async_copy(didx(1), dblk.at[1], isem)
    for i in range(A_CPG):
        pltpu.async_copy(g_hbm.at[sblk.at[0, i]], rows.at[0, i], gsem)

    def body(g, _):
        p = lax.rem(g, 2)
        q = lax.rem(g + 1, 2)
        for i in range(A_CPG):
            pltpu.make_async_copy(g_hbm.at[sblk.at[p, i]], rows.at[p, i],
                                  gsem).wait()
        for i in range(A_CPG):
            for t in range(CH // 16):
                d16 = dblk[p, i, pl.ds(16 * t, 16)]
                m = (d16 >= base) & (d16 < base + HALF)
                dloc[p, i, pl.ds(16 * t, 16)] = jnp.where(m, d16 - base, HALF)
        for i in range(A_CPG):
            pltpu.async_copy(rows.at[p, i], acc.at[dloc.at[p, i]], ssem,
                             add=True)

        @pl.when(g > 0)
        def _():
            for i in range(A_CPG):
                pltpu.make_async_copy(rows.at[q, i], acc.at[dloc.at[q, i]],
                                      ssem).wait()

        @pl.when(g < A_NG - 1)
        def _():
            pltpu.make_async_copy(sidx(g + 1), sblk.at[q], isem).wait()
            pltpu.make_async_copy(didx(g + 1), dblk.at[q], isem).wait()

        @pl.when(g < A_NG - 2)
        def _():
            pltpu.async_copy(sidx(g + 2), sblk.at[p], isem)
            pltpu.async_copy(didx(g + 2), dblk.at[p], isem)

        @pl.when(g < A_NG - 1)
        def _():
            for i in range(A_CPG):
                pltpu.async_copy(g_hbm.at[sblk.at[q, i]], rows.at[q, i], gsem)

        return 0

    lax.fori_loop(0, A_NG, body, 0)
    pl_last = lax.rem(A_NG - 1, 2)
    for i in range(A_CPG):
        pltpu.make_async_copy(rows.at[pl_last, i],
                              acc.at[dloc.at[pl_last, i]], ssem).wait()

    plsc.subcore_barrier()

    def drain(k, _):
        m = s + 16 * k

        @pl.when(m < HALF // 128)
        def _():
            pltpu.sync_copy(acc.at[pl.ds(m * 128, 128)], rows.at[0, 0])
            pltpu.sync_copy(rows.at[0, 0],
                            out_hbm.at[pl.ds(c * HALF + m * 128, 128)])

        @pl.when(m == HALF // 128)
        def _():
            pltpu.sync_copy(acc.at[pl.ds(m * 128, 64)],
                            rows.at[0, 0, pl.ds(0, 64)])
            pltpu.sync_copy(rows.at[0, 0, pl.ds(0, 64)],
                            out_hbm.at[pl.ds(c * HALF + m * 128, 64)])

        return 0

    lax.fori_loop(0, (HALF // 128 + 16) // 16, drain, 0)


_SC_MESH = plsc.VectorSubcoreMesh(core_axis_name="c", subcore_axis_name="s")
_SC_PARAMS = pltpu.CompilerParams(use_tc_tiling_on_sc=False)

_deg_call = pl.kernel(
    _deg_body,
    out_type=jax.ShapeDtypeStruct((2 * NP,), jnp.float32),
    mesh=_SC_MESH,
    scratch_types=[
        pltpu.VMEM((CH,), jnp.float32),
        pltpu.VMEM((3136,), jnp.float32),
        pltpu.VMEM((3, D_CPG, CH), jnp.int32),
        pltpu.VMEM_SHARED((NP,), jnp.float32),
        pltpu.SemaphoreType.DMA,
        pltpu.SemaphoreType.DMA,
    ],
    compiler_params=_SC_PARAMS,
)

_agg_call = pl.kernel(
    _agg_body,
    out_type=jax.ShapeDtypeStruct((NP, H2), jnp.float32),
    mesh=_SC_MESH,
    scratch_types=[
        pltpu.VMEM((2, A_CPG, CH), jnp.int32),
        pltpu.VMEM((2, A_CPG, CH), jnp.int32),
        pltpu.VMEM((2, A_CPG, CH), jnp.int32),
        pltpu.VMEM((2, A_CPG, CH, H2), jnp.float32),
        pltpu.VMEM_SHARED((ACC_ROWS, H2), jnp.float32),
        pltpu.SemaphoreType.DMA,
        pltpu.SemaphoreType.DMA,
        pltpu.SemaphoreType.DMA,
    ],
    compiler_params=_SC_PARAMS,
)


def kernel(x, edge_index, batch, W_in, b_in, conv_W, conv_b, gn_w, gn_b, gn_ms,
           Wc1, bc1, Wc2, bc2, Wc3, bc3, Ws1, bs1, Ws2, bs2):
    src2 = edge_index[0].reshape(EC, CH)
    dst2 = edge_index[1].reshape(EC, CH)
    src2 = jnp.pad(src2, ((0, ECP - EC), (0, 0)))
    dst2 = jnp.pad(dst2, ((0, ECP - EC), (0, 0)), constant_values=DUMMY_DST)

    degpart = _deg_call(dst2)
    deg = degpart[:N] + degpart[NP:NP + N] + 1.0
    dinv = lax.rsqrt(deg)

    h = x @ W_in + b_in
    for l in range(3):
        hlin = h @ conv_W[l]
        g = dinv[:, None] * hlin
        gp = jnp.pad(g, ((0, NP - N), (0, 0)))
        agg = jnp.concatenate(
            [_agg_call(gp[:, :H2], src2, dst2),
             _agg_call(gp[:, H2:], src2, dst2)], axis=1)[:N]
        t = dinv[:, None] * (agg + g) + conv_b[l]
        mean = jnp.mean(t, axis=0)
        sub = t - mean * gn_ms[l]
        var = jnp.mean(sub * sub, axis=0)
        t = gn_w[l] * sub / jnp.sqrt(var + 1e-5) + gn_b[l]
        h = h + jax.nn.relu(t)

    graph_emb = jnp.mean(h, axis=0, keepdims=True)
    z = jax.nn.relu(graph_emb @ Wc1 + bc1)
    z = jax.nn.relu(z @ Wc2 + bc2)
    logits = z @ Wc3 + bc3
    probs = jax.nn.softmax(logits, axis=-1)
    ns = jax.nn.relu(h @ Ws1 + bs1)
    node_hub_scores = jax.nn.sigmoid(ns @ Ws2 + bs2)[:, 0]
    return logits, probs, graph_emb, h, node_hub_scores


# speedup vs baseline: 1.2388x; 1.2388x over previous
import jax
import jax.numpy as jnp
from jax import lax
from jax.experimental import pallas as pl
from jax.experimental.pallas import tpu as pltpu
from jax.experimental.pallas import tpu_sc as plsc

N = 50000
E = 800000
D = 128
H = 64
NP = 50048
HALF = NP // 2
ACC_ROWS = HALF + 64
H2 = 64
CH = 128
EC = E // CH
ECP = 6400
DUMMY_DST = NP - 8

A_CPT = ECP // 16
A_CPG = 1
A_NG = A_CPT // A_CPG

D_CPT = ECP // 32
D_CPG = 8
D_NG = D_CPT // D_CPG


def _deg_body(dst_hbm, out_hbm, ones_v, zbuf, dblk, acc, ssem, isem):
    c = lax.axis_index("c")
    s = lax.axis_index("s")
    zero16 = jnp.zeros((16,), jnp.float32)
    one16 = jnp.ones((16,), jnp.float32)
    for k in range(CH // 16):
        ones_v[pl.ds(16 * k, 16)] = one16

    def zb(i, _):
        zbuf[pl.ds(i * 16, 16)] = zero16
        return 0

    lax.fori_loop(0, 196, zb, 0)
    tpr = NP // 16
    pltpu.sync_copy(zbuf.at[pl.ds(0, tpr)], acc.at[pl.ds(s * tpr, tpr)])
    plsc.subcore_barrier()

    start = c * (ECP // 2) + s * D_CPT

    def idx_src(g):
        return dst_hbm.at[pl.ds(start + D_CPG * g, D_CPG)]

    pltpu.sync_copy(idx_src(0), dblk.at[0])
    pltpu.async_copy(idx_src(1), dblk.at[1], isem)

    def body(g, _):
        r = lax.rem(g, 3)
        rp = lax.rem(g + 2, 3)
        for i in range(D_CPG):
            pltpu.async_copy(ones_v, acc.at[dblk.at[r, i]], ssem, add=True)

        @pl.when(g > 0)
        def _():
            for i in range(D_CPG):
                pltpu.make_async_copy(ones_v, acc.at[dblk.at[rp, i]], ssem).wait()

        @pl.when(g < D_NG - 1)
        def _():
            pltpu.make_async_copy(idx_src(g + 1), dblk.at[lax.rem(g + 1, 3)],
                                  isem).wait()

        @pl.when(g < D_NG - 2)
        def _():
            pltpu.async_copy(idx_src(g + 2), dblk.at[rp], isem)

        return 0

    lax.fori_loop(0, D_NG, body, 0)
    rl = lax.rem(D_NG - 1, 3)
    for i in range(D_CPG):
        pltpu.make_async_copy(ones_v, acc.at[dblk.at[rl, i]], ssem).wait()

    plsc.subcore_barrier()
    pltpu.sync_copy(acc.at[pl.ds(s * tpr, tpr)], zbuf.at[pl.ds(0, tpr)])
    pltpu.sync_copy(zbuf.at[pl.ds(0, tpr)],
                    out_hbm.at[pl.ds(c * NP + s * tpr, tpr)])


def _agg_body(g_hbm, src_hbm, dst_hbm, out_hbm, sblk, dblk, dloc, rows, acc,
              gsem, ssem, isem):
    c = lax.axis_index("c")
    s = lax.axis_index("s")
    base = c * HALF
    zero16 = jnp.zeros((16,), jnp.float32)

    def zb(i, _):
        for k in range(H2 // 16):
            rows[0, 0, i, pl.ds(16 * k, 16)] = zero16
        return 0

    lax.fori_loop(0, CH, zb, 0)
    nchunks = ACC_ROWS // 128

    def zc(k, _):
        m = s + 16 * k

        @pl.when(m < nchunks)
        def _():
            pltpu.sync_copy(rows.at[0, 0], acc.at[pl.ds(m * 128, 128)])

        return 0

    lax.fori_loop(0, (nchunks + 15) // 16, zc, 0)
    plsc.subcore_barrier()

    start = s * A_CPT

    def sidx(g):
        return src_hbm.at[pl.ds(start + A_CPG * g, A_CPG)]

    def didx(g):
        return dst_hbm.at[pl.ds(start + A_CPG * g, A_CPG)]

    pltpu.sync_copy(sidx(0), sblk.at[0])
    pltpu.sync_copy(didx(0), dblk.at[0])
    pltpu.async_copy(sidx(1), sblk.at[1], isem)
    pltpu.async_copy(didx(1), dblk.at[1], isem)
    for i in range(A_CPG):
        pltpu.async_copy(g_hbm.at[sblk.at[0, i]], rows.at[0, i], gsem)

    def body(g, _):
        p = lax.rem(g, 2)
        q = lax.rem(g + 1, 2)
        for i in range(A_CPG):
            pltpu.make_async_copy(g_hbm.at[sblk.at[p, i]], rows.at[p, i],
                                  gsem).wait()
        for i in range(A_CPG):
            for t in range(CH // 16):
                d16 = dblk[p, i, pl.ds(16 * t, 16)]
                m = (d16 >= base) & (d16 < base + HALF)
                dloc[p, i, pl.ds(16 * t, 16)] = jnp.where(m, d16 - base, HALF)
        for i in range(A_CPG):
            pltpu.async_copy(rows.at[p, i], acc.at[dloc.at[p, i]], ssem,
                             add=True)

        @pl.when(g > 0)
        def _():
            for i in range(A_CPG):
                pltpu.make_async_copy(rows.at[q, i], acc.at[dloc.at[q, i]],
                                      ssem).wait()

        @pl.when(g < A_NG - 1)
        def _():
            pltpu.make_async_copy(sidx(g + 1), sblk.at[q], isem).wait()
            pltpu.make_async_copy(didx(g + 1), dblk.at[q], isem).wait()

        @pl.when(g < A_NG - 2)
        def _():
            pltpu.async_copy(sidx(g + 2), sblk.at[p], isem)
            pltpu.async_copy(didx(g + 2), dblk.at[p], isem)

        @pl.when(g < A_NG - 1)
        def _():
            for i in range(A_CPG):
                pltpu.async_copy(g_hbm.at[sblk.at[q, i]], rows.at[q, i], gsem)

        return 0

    lax.fori_loop(0, A_NG, body, 0)
    pl_last = lax.rem(A_NG - 1, 2)
    for i in range(A_CPG):
        pltpu.make_async_copy(rows.at[pl_last, i],
                              acc.at[dloc.at[pl_last, i]], ssem).wait()

    plsc.subcore_barrier()

    def drain(k, _):
        m = s + 16 * k

        @pl.when(m < HALF // 128)
        def _():
            pltpu.sync_copy(acc.at[pl.ds(m * 128, 128)], rows.at[0, 0])
            pltpu.sync_copy(rows.at[0, 0],
                            out_hbm.at[pl.ds(c * HALF + m * 128, 128)])

        @pl.when(m == HALF // 128)
        def _():
            pltpu.sync_copy(acc.at[pl.ds(m * 128, 64)],
                            rows.at[0, 0, pl.ds(0, 64)])
            pltpu.sync_copy(rows.at[0, 0, pl.ds(0, 64)],
                            out_hbm.at[pl.ds(c * HALF + m * 128, 64)])

        return 0

    lax.fori_loop(0, (HALF // 128 + 16) // 16, drain, 0)


_SC_MESH = plsc.VectorSubcoreMesh(core_axis_name="c", subcore_axis_name="s")
_SC_PARAMS = pltpu.CompilerParams(use_tc_tiling_on_sc=False)

_deg_call = pl.kernel(
    _deg_body,
    out_type=jax.ShapeDtypeStruct((2 * NP,), jnp.float32),
    mesh=_SC_MESH,
    scratch_types=[
        pltpu.VMEM((CH,), jnp.float32),
        pltpu.VMEM((3136,), jnp.float32),
        pltpu.VMEM((3, D_CPG, CH), jnp.int32),
        pltpu.VMEM_SHARED((NP,), jnp.float32),
        pltpu.SemaphoreType.DMA,
        pltpu.SemaphoreType.DMA,
    ],
    compiler_params=_SC_PARAMS,
)

_agg_call = pl.kernel(
    _agg_body,
    out_type=jax.ShapeDtypeStruct((NP, H2), jnp.float32),
    mesh=_SC_MESH,
    scratch_types=[
        pltpu.VMEM((2, A_CPG, CH), jnp.int32),
        pltpu.VMEM((2, A_CPG, CH), jnp.int32),
        pltpu.VMEM((2, A_CPG, CH), jnp.int32),
        pltpu.VMEM((2, A_CPG, CH, H2), jnp.float32),
        pltpu.VMEM_SHARED((ACC_ROWS, H2), jnp.float32),
        pltpu.SemaphoreType.DMA,
        pltpu.SemaphoreType.DMA,
        pltpu.SemaphoreType.DMA,
    ],
    compiler_params=_SC_PARAMS,
)


BR = 3128
NBLK = NP // BR


def _dinv_body(d0_ref, d1_ref, o_ref):
    o_ref[...] = lax.rsqrt(d0_ref[...] + d1_ref[...] + 1.0)


def _dinv_tc(d0, d1):
    return pl.pallas_call(
        _dinv_body,
        out_shape=jax.ShapeDtypeStruct((NP // 128, 128), jnp.float32),
    )(d0, d1)


def _fused_in_body(x_ref, Wi_ref, bi_ref, W0_ref, dv_ref, h_ref, g_ref):
    h = jnp.dot(x_ref[...], Wi_ref[...],
                preferred_element_type=jnp.float32) + bi_ref[...]
    h_ref[...] = h
    g_ref[...] = dv_ref[...] * jnp.dot(h, W0_ref[...],
                                       preferred_element_type=jnp.float32)


def _fused_in_tc(xp, W_in, b_in, W0, dinvc):
    return pl.pallas_call(
        _fused_in_body,
        grid=(NBLK,),
        in_specs=[
            pl.BlockSpec((BR, D), lambda i: (i, 0)),
            pl.BlockSpec((D, H), lambda i: (0, 0)),
            pl.BlockSpec((1, H), lambda i: (0, 0)),
            pl.BlockSpec((H, H), lambda i: (0, 0)),
            pl.BlockSpec((BR, 1), lambda i: (i, 0)),
        ],
        out_specs=[
            pl.BlockSpec((BR, H), lambda i: (i, 0)),
            pl.BlockSpec((BR, H), lambda i: (i, 0)),
        ],
        out_shape=[
            jax.ShapeDtypeStruct((NP, H), jnp.float32),
            jax.ShapeDtypeStruct((NP, H), jnp.float32),
        ],
    )(xp, W_in, b_in, W0, dinvc)


def _post_body(agg_ref, g_ref, dv_ref, b_ref, t_ref, st_ref):
    i = pl.program_id(0)
    t = dv_ref[...] * (agg_ref[...] + g_ref[...]) + b_ref[...]
    t_ref[...] = t
    rows = i * BR + lax.broadcasted_iota(jnp.int32, (BR, 1), 0)
    tm = jnp.where(rows < N, t, 0.0)

    @pl.when(i == 0)
    def _():
        st_ref[...] = jnp.zeros_like(st_ref)

    st_ref[0, :] += jnp.sum(tm, axis=0)
    st_ref[1, :] += jnp.sum(tm * t, axis=0)


def _post_tc(agg, g, dinvc, b):
    return pl.pallas_call(
        _post_body,
        grid=(NBLK,),
        in_specs=[
            pl.BlockSpec((BR, H), lambda i: (i, 0)),
            pl.BlockSpec((BR, H), lambda i: (i, 0)),
            pl.BlockSpec((BR, 1), lambda i: (i, 0)),
            pl.BlockSpec((1, H), lambda i: (0, 0)),
        ],
        out_specs=[
            pl.BlockSpec((BR, H), lambda i: (i, 0)),
            pl.BlockSpec((2, H), lambda i: (0, 0)),
        ],
        out_shape=[
            jax.ShapeDtypeStruct((NP, H), jnp.float32),
            jax.ShapeDtypeStruct((2, H), jnp.float32),
        ],
    )(agg, g, dinvc, b)


def _norm_block(t, st, gw, gb, gm):
    mean = st[0:1, :] * (1.0 / N)
    var = st[1:2, :] * (1.0 / N) - mean * mean * gm * (2.0 - gm)
    rstd = lax.rsqrt(var + 1e-5)
    sub = t - mean * gm
    return jax.nn.relu(gw * sub * rstd + gb)


def _next_body(t_ref, st_ref, h_ref, dv_ref, Wn_ref, gw_ref, gb_ref, gm_ref,
               ho_ref, go_ref):
    hn = h_ref[...] + _norm_block(t_ref[...], st_ref[...], gw_ref[...],
                                  gb_ref[...], gm_ref[...])
    ho_ref[...] = hn
    go_ref[...] = dv_ref[...] * jnp.dot(hn, Wn_ref[...],
                                        preferred_element_type=jnp.float32)


def _next_tc(t, st, h, dinvc, Wn, gw, gb, gm):
    return pl.pallas_call(
        _next_body,
        grid=(NBLK,),
        in_specs=[
            pl.BlockSpec((BR, H), lambda i: (i, 0)),
            pl.BlockSpec((2, H), lambda i: (0, 0)),
            pl.BlockSpec((BR, H), lambda i: (i, 0)),
            pl.BlockSpec((BR, 1), lambda i: (i, 0)),
            pl.BlockSpec((H, H), lambda i: (0, 0)),
            pl.BlockSpec((1, H), lambda i: (0, 0)),
            pl.BlockSpec((1, H), lambda i: (0, 0)),
            pl.BlockSpec((1, H), lambda i: (0, 0)),
        ],
        out_specs=[
            pl.BlockSpec((BR, H), lambda i: (i, 0)),
            pl.BlockSpec((BR, H), lambda i: (i, 0)),
        ],
        out_shape=[
            jax.ShapeDtypeStruct((NP, H), jnp.float32),
            jax.ShapeDtypeStruct((NP, H), jnp.float32),
        ],
    )(t, st, h, dinvc, Wn, gw, gb, gm)


def _last_body(t_ref, st_ref, h_ref, gw_ref, gb_ref, gm_ref, Ws1_ref, bs1_ref,
               Ws2_ref, bs2_ref, ho_ref, hs_ref, sc_ref):
    i = pl.program_id(0)
    hn = h_ref[...] + _norm_block(t_ref[...], st_ref[...], gw_ref[...],
                                  gb_ref[...], gm_ref[...])
    ho_ref[...] = hn
    rows = i * BR + lax.broadcasted_iota(jnp.int32, (BR, 1), 0)

    @pl.when(i == 0)
    def _():
        hs_ref[...] = jnp.zeros_like(hs_ref)

    hs_ref[0, :] += jnp.sum(jnp.where(rows < N, hn, 0.0), axis=0)
    ns = jax.nn.relu(jnp.dot(hn, Ws1_ref[...],
                             preferred_element_type=jnp.float32) + bs1_ref[...])
    sc_ref[...] = jax.nn.sigmoid(
        jnp.dot(ns, Ws2_ref[...], preferred_element_type=jnp.float32)
        + bs2_ref[...])


def _last_tc(t, st, h, gw, gb, gm, Ws1, bs1, Ws2, bs2):
    return pl.pallas_call(
        _last_body,
        grid=(NBLK,),
        in_specs=[
            pl.BlockSpec((BR, H), lambda i: (i, 0)),
            pl.BlockSpec((2, H), lambda i: (0, 0)),
            pl.BlockSpec((BR, H), lambda i: (i, 0)),
            pl.BlockSpec((1, H), lambda i: (0, 0)),
            pl.BlockSpec((1, H), lambda i: (0, 0)),
            pl.BlockSpec((1, H), lambda i: (0, 0)),
            pl.BlockSpec((H, H // 2), lambda i: (0, 0)),
            pl.BlockSpec((1, H // 2), lambda i: (0, 0)),
            pl.BlockSpec((H // 2, 1), lambda i: (0, 0)),
            pl.BlockSpec((1, 1), lambda i: (0, 0)),
        ],
        out_specs=[
            pl.BlockSpec((BR, H), lambda i: (i, 0)),
            pl.BlockSpec((1, H), lambda i: (0, 0)),
            pl.BlockSpec((BR, 1), lambda i: (i, 0)),
        ],
        out_shape=[
            jax.ShapeDtypeStruct((NP, H), jnp.float32),
            jax.ShapeDtypeStruct((1, H), jnp.float32),
            jax.ShapeDtypeStruct((NP, 1), jnp.float32),
        ],
    )(t, st, h, gw, gb, gm, Ws1, bs1, Ws2, bs2)


def _heads_body(hs_ref, Wc1_ref, bc1_ref, Wc2_ref, bc2_ref, Wc3_ref, bc3_ref,
                lg_ref, pr_ref, ge_ref):
    ge = hs_ref[...] * (1.0 / N)
    ge_ref[...] = ge
    z = jax.nn.relu(jnp.dot(ge, Wc1_ref[...],
                            preferred_element_type=jnp.float32) + bc1_ref[...])
    z = jax.nn.relu(jnp.dot(z, Wc2_ref[...],
                            preferred_element_type=jnp.float32) + bc2_ref[...])
    lg = jnp.dot(z, Wc3_ref[...],
                 preferred_element_type=jnp.float32) + bc3_ref[...]
    lg_ref[...] = lg
    m = jnp.max(lg, axis=1, keepdims=True)
    e = jnp.exp(lg - m)
    pr_ref[...] = e / jnp.sum(e, axis=1, keepdims=True)


def _heads_tc(hs, Wc1, bc1, Wc2, bc2, Wc3, bc3):
    return pl.pallas_call(
        _heads_body,
        out_shape=[
            jax.ShapeDtypeStruct((1, 2), jnp.float32),
            jax.ShapeDtypeStruct((1, 2), jnp.float32),
            jax.ShapeDtypeStruct((1, H), jnp.float32),
        ],
    )(hs, Wc1, bc1, Wc2, bc2, Wc3, bc3)


def kernel(x, edge_index, batch, W_in, b_in, conv_W, conv_b, gn_w, gn_b, gn_ms,
           Wc1, bc1, Wc2, bc2, Wc3, bc3, Ws1, bs1, Ws2, bs2):
    src2 = edge_index[0].reshape(EC, CH)
    dst2 = edge_index[1].reshape(EC, CH)
    src2 = jnp.pad(src2, ((0, ECP - EC), (0, 0)))
    dst2 = jnp.pad(dst2, ((0, ECP - EC), (0, 0)), constant_values=DUMMY_DST)

    degpart = _deg_call(dst2)
    dp = degpart.reshape(2, NP // 128, 128)
    dinvc = _dinv_tc(dp[0], dp[1]).reshape(NP, 1)

    xp = jnp.pad(x, ((0, NP - N), (0, 0)))
    h, g = _fused_in_tc(xp, W_in, b_in.reshape(1, H), conv_W[0], dinvc)
    for l in range(3):
        agg = _agg_call(g, src2, dst2)
        t, st = _post_tc(agg, g, dinvc, conv_b[l].reshape(1, H))
        gw = gn_w[l].reshape(1, H)
        gb = gn_b[l].reshape(1, H)
        gm = gn_ms[l].reshape(1, H)
        if l < 2:
            h, g = _next_tc(t, st, h, dinvc, conv_W[l + 1], gw, gb, gm)
        else:
            h, hs, sc = _last_tc(t, st, h, gw, gb, gm, Ws1.reshape(H, H // 2),
                                 bs1.reshape(1, H // 2), Ws2.reshape(H // 2, 1),
                                 bs2.reshape(1, 1))
    logits, probs, graph_emb = _heads_tc(hs, Wc1, bc1.reshape(1, H),
                                         Wc2, bc2.reshape(1, H // 2),
                                         Wc3, bc3.reshape(1, 2))
    return logits, probs, graph_emb, h[:N], sc[:N, 0]
